# baseline (speedup 1.0000x reference)
import functools

import jax
import jax.numpy as jnp
from jax import lax
from jax.experimental import pallas as pl
from jax.experimental.pallas import tpu as pltpu
from jax.experimental.pallas import tpu_sc as plsc

_L = 16
_N = 12


def _compact_body(t2_hbm, st_hbm, out_hbm, t2_v, st_v, out_v):
    @pl.when((lax.axis_index("c") == 0) & (lax.axis_index("s") == 0))
    def _():
        pltpu.sync_copy(t2_hbm, t2_v)
        pltpu.sync_copy(st_hbm, st_v)
        t = t2_v[...]
        s = st_v[...]
        lane = lax.iota(jnp.int32, _L)
        m = (t < 1.0) & (lane < _N)
        ranks = plsc.cumsum(m.astype(jnp.int32)) - 1
        fill = plsc.load_gather(st_v, [jnp.zeros((_L,), jnp.int32)])
        out_v[...] = fill
        plsc.store_scatter(out_v, [ranks], s, mask=m)
        pltpu.sync_copy(out_v, out_hbm)


def kernel(t2, stored_tensor):
    t2f = jnp.pad(t2.reshape(-1), (0, _L - _N), constant_values=1.0)
    stf = jnp.pad(stored_tensor.reshape(-1), (0, _L - _N))
    mesh = plsc.VectorSubcoreMesh(core_axis_name="c", subcore_axis_name="s")
    run = pl.kernel(
        _compact_body,
        mesh=mesh,
        out_type=jax.ShapeDtypeStruct((_L,), jnp.float32),
        scratch_types=[
            pltpu.VMEM((_L,), jnp.float32),
            pltpu.VMEM((_L,), jnp.float32),
            pltpu.VMEM((_L,), jnp.float32),
        ],
        compiler_params=pltpu.CompilerParams(needs_layout_passes=False),
    )
    return run(t2f, stf)[:_N]


# speedup vs baseline: 1.1155x; 1.1155x over previous
import jax
import jax.numpy as jnp
from jax.experimental import pallas as pl
from jax.experimental.pallas import tpu as pltpu
from jax.experimental.pallas import tpu_sc as plsc

_L = 16
_N = 12


def _compact_body(in_hbm, out_hbm, in_v, out_v):
    pltpu.sync_copy(in_hbm, in_v)
    t = in_v[pl.ds(0, _L)]
    s = in_v[pl.ds(_L, _L)]
    m = t < 1.0
    fill = plsc.load_gather(in_v, [jnp.full((_L,), _L, jnp.int32)])
    out_v[...] = fill
    plsc.store_compressed(out_v.at[...], s, mask=m)
    pltpu.sync_copy(out_v, out_hbm)


def kernel(t2, stored_tensor):
    t2f = jnp.pad(t2.reshape(-1), (0, _L - _N), constant_values=1.0)
    stf = jnp.pad(stored_tensor.reshape(-1), (0, _L - _N))
    packed = jnp.concatenate([t2f, stf])
    mesh = plsc.VectorSubcoreMesh(
        core_axis_name="c", subcore_axis_name="s", num_cores=1, num_subcores=1
    )
    run = pl.kernel(
        _compact_body,
        mesh=mesh,
        out_type=jax.ShapeDtypeStruct((_L,), jnp.float32),
        scratch_types=[
            pltpu.VMEM((2 * _L,), jnp.float32),
            pltpu.VMEM((_L,), jnp.float32),
        ],
        compiler_params=pltpu.CompilerParams(needs_layout_passes=False),
    )
    return run(packed)[:_N]
